# baseline (speedup 1.0000x reference)
import functools

import jax
import jax.numpy as jnp
from jax import lax
from jax.experimental import pallas as pl
from jax.experimental.pallas import tpu as pltpu
from jax.experimental.pallas import tpu_sc as plsc

NC = 2
NS = 16
NW = NC * NS
CH = 16


def _pad_rows(rows, n_pad_to):
    extra = n_pad_to - rows.shape[0]
    if extra == 0:
        return rows
    reps = -(-extra // rows.shape[0])
    return jnp.concatenate([rows, jnp.tile(rows, reps)[:extra]])


def _sc_body(nch_c, nch_z, row, xf, csrc, cdst, zdst, out,
             csrc_v, cdst_v, zdst_v, buf, zbuf, sem):
    wid = lax.axis_index("s") * NC + lax.axis_index("c")

    pltpu.sync_copy(csrc.at[wid], csrc_v)
    pltpu.sync_copy(cdst.at[wid], cdst_v)
    pltpu.sync_copy(zdst.at[wid], zdst_v)

    zeros16 = jnp.zeros((16,), jnp.float32)

    def zfill(t, carry):
        i = t // (row // 16)
        j = t % (row // 16)
        zbuf[i, pl.ds(j * 16, 16)] = zeros16
        return carry

    lax.fori_loop(0, CH * (row // 16), zfill, 0)

    def cbody(c, carry):
        pltpu.async_copy(xf.at[csrc_v.at[c]], buf, sem).wait()
        pltpu.async_copy(buf, out.at[cdst_v.at[c]], sem).wait()
        return carry

    lax.fori_loop(0, nch_c, cbody, 0)

    def zbody(c, carry):
        pltpu.async_copy(zbuf, out.at[zdst_v.at[c]], sem).wait()
        return carry

    lax.fori_loop(0, nch_z, zbody, 0)


def kernel(x, in_channels, out_channels):
    B, C_in, H, W = x.shape
    C_out = out_channels.shape[0]
    P = H * W
    row = 1024 if P % 1024 == 0 else P
    G = P // row

    nzo = out_channels != 0
    in_pos = jnp.where(in_channels != 0, size=C_in, fill_value=0)[0]
    out_pos = jnp.where(nzo, size=C_in, fill_value=0)[0]
    inv_pos = jnp.where(~nzo, size=C_out - C_in, fill_value=0)[0]

    b = jnp.arange(B, dtype=jnp.int32)
    g = jnp.arange(G, dtype=jnp.int32)

    def plane_rows(chan_pos, stride):
        planes = b[:, None] * stride + chan_pos[None, :].astype(jnp.int32)
        return (planes[:, :, None] * G + g[None, None, :]).reshape(-1)

    src_rows = plane_rows(in_pos, C_in)
    dst_rows = plane_rows(out_pos, C_out)
    zdst_rows = plane_rows(inv_pos, C_out)

    nch_c = -(-src_rows.shape[0] // (NW * CH))
    nch_z = -(-zdst_rows.shape[0] // (NW * CH))
    csrc = _pad_rows(src_rows, NW * nch_c * CH).reshape(NW, nch_c, CH)
    cdst = _pad_rows(dst_rows, NW * nch_c * CH).reshape(NW, nch_c, CH)
    zdst = _pad_rows(zdst_rows, NW * nch_z * CH).reshape(NW, nch_z, CH)

    xf = x.reshape(B * C_in * G, row)

    mesh = plsc.VectorSubcoreMesh(
        core_axis_name="c", subcore_axis_name="s",
        num_cores=NC, num_subcores=NS)
    body = functools.partial(_sc_body, nch_c, nch_z, row)
    out = pl.kernel(
        body,
        out_type=jax.ShapeDtypeStruct((B * C_out * G, row), jnp.float32),
        mesh=mesh,
        scratch_types=[
            pltpu.VMEM((nch_c, CH), jnp.int32),
            pltpu.VMEM((nch_c, CH), jnp.int32),
            pltpu.VMEM((nch_z, CH), jnp.int32),
            pltpu.VMEM((CH, row), jnp.float32),
            pltpu.VMEM((CH, row), jnp.float32),
            pltpu.SemaphoreType.DMA,
        ],
    )(xf, csrc, cdst, zdst)
    return out.reshape(B, C_out, H, W)


# speedup vs baseline: 1.0606x; 1.0606x over previous
import functools

import jax
import jax.numpy as jnp
from jax import lax
from jax.experimental import pallas as pl
from jax.experimental.pallas import tpu as pltpu
from jax.experimental.pallas import tpu_sc as plsc

NC = 2
NS = 16
NW = NC * NS
CH = 16
ND = 4
ZW = 12


def _pad_rows(rows, n_pad_to):
    extra = n_pad_to - rows.shape[0]
    if extra == 0:
        return rows
    reps = -(-extra // rows.shape[0])
    return jnp.concatenate([rows, jnp.tile(rows, reps)[:extra]])


def _sc_body(nch_c, nch_z, row, xf, csrc, cdst, zdst, out,
             csrc_v, cdst_v, zdst_v, buf, zbuf, sem_z, sems_g, sems_s):
    wid = lax.axis_index("s") * NC + lax.axis_index("c")

    pltpu.sync_copy(csrc.at[wid], csrc_v)
    pltpu.sync_copy(cdst.at[wid], cdst_v)
    pltpu.sync_copy(zdst.at[wid], zdst_v)

    zeros16 = jnp.zeros((16,), jnp.float32)

    def zfill(t, carry):
        i = t // (row // 16)
        j = t % (row // 16)
        zbuf[i, pl.ds(j * 16, 16)] = zeros16
        return carry

    lax.fori_loop(0, CH * (row // 16), zfill, 0)

    def zero_fire(c):
        pltpu.async_copy(zbuf, out.at[zdst_v.at[c]], sem_z)

    def zero_drain(c):
        pltpu.make_async_copy(zbuf, out.at[zdst_v.at[c]], sem_z).wait()

    def gather_fire(c, j):
        pltpu.async_copy(xf.at[csrc_v.at[c]], buf.at[j], sems_g[j])

    def gather_wait(c, j):
        pltpu.make_async_copy(xf.at[csrc_v.at[c]], buf.at[j], sems_g[j]).wait()

    def scatter_fire(c, j):
        pltpu.async_copy(buf.at[j], out.at[cdst_v.at[c]], sems_s[j])

    def scatter_wait(c, j):
        pltpu.make_async_copy(buf.at[j], out.at[cdst_v.at[c]], sems_s[j]).wait()

    def zloop(c, carry):
        zero_fire(c)

        @pl.when(c >= ZW)
        def _():
            zero_drain(c - ZW)
        return carry

    lax.fori_loop(0, nch_z, zloop, 0)

    nblk = nch_c // ND
    rem = nch_c % ND

    def blk_body(blk, carry):
        c0 = blk * ND
        for j in range(ND):
            @pl.when(blk > 0)
            def _(j=j):
                scatter_wait(c0 - ND + j, j)
            gather_fire(c0 + j, j)
        for j in range(ND):
            gather_wait(c0 + j, j)
            scatter_fire(c0 + j, j)
        return carry

    if nblk > 0:
        lax.fori_loop(0, nblk, blk_body, 0)

    for j in range(rem):
        c = nblk * ND + j
        if nblk > 0:
            scatter_wait(c - ND, j)
        gather_fire(c, j)
        gather_wait(c, j)
        scatter_fire(c, j)

    for j in range(ND if nblk > 0 else rem):
        cc = nblk * ND + j if j < rem else (nblk - 1) * ND + j
        scatter_wait(cc, j)

    def zdrain_loop(c, carry):
        zero_drain(c)
        return carry

    lax.fori_loop(max(nch_z - ZW, 0), nch_z, zdrain_loop, 0)


def kernel(x, in_channels, out_channels):
    B, C_in, H, W = x.shape
    C_out = out_channels.shape[0]
    P = H * W
    row = 1024 if P % 1024 == 0 else P
    G = P // row

    nzo = out_channels != 0
    in_pos = jnp.where(in_channels != 0, size=C_in, fill_value=0)[0]
    out_pos = jnp.where(nzo, size=C_in, fill_value=0)[0]
    inv_pos = jnp.where(~nzo, size=C_out - C_in, fill_value=0)[0]

    b = jnp.arange(B, dtype=jnp.int32)
    g = jnp.arange(G, dtype=jnp.int32)

    def plane_rows(chan_pos, stride):
        planes = b[:, None] * stride + chan_pos[None, :].astype(jnp.int32)
        return (planes[:, :, None] * G + g[None, None, :]).reshape(-1)

    src_rows = plane_rows(in_pos, C_in)
    dst_rows = plane_rows(out_pos, C_out)
    zdst_rows = plane_rows(inv_pos, C_out)

    nch_c = -(-src_rows.shape[0] // (NW * CH))
    nch_z = -(-zdst_rows.shape[0] // (NW * CH))
    csrc = _pad_rows(src_rows, NW * nch_c * CH).reshape(NW, nch_c, CH)
    cdst = _pad_rows(dst_rows, NW * nch_c * CH).reshape(NW, nch_c, CH)
    zdst = _pad_rows(zdst_rows, NW * nch_z * CH).reshape(NW, nch_z, CH)

    xf = x.reshape(B * C_in * G, row)

    mesh = plsc.VectorSubcoreMesh(
        core_axis_name="c", subcore_axis_name="s",
        num_cores=NC, num_subcores=NS)
    body = functools.partial(_sc_body, nch_c, nch_z, row)
    out = pl.kernel(
        body,
        out_type=jax.ShapeDtypeStruct((B * C_out * G, row), jnp.float32),
        mesh=mesh,
        scratch_types=[
            pltpu.VMEM((nch_c, CH), jnp.int32),
            pltpu.VMEM((nch_c, CH), jnp.int32),
            pltpu.VMEM((nch_z, CH), jnp.int32),
            pltpu.VMEM((ND, CH, row), jnp.float32),
            pltpu.VMEM((CH, row), jnp.float32),
            pltpu.SemaphoreType.DMA,
            [pltpu.SemaphoreType.DMA] * ND,
            [pltpu.SemaphoreType.DMA] * ND,
        ],
    )(xf, csrc, cdst, zdst)
    return out.reshape(B, C_out, H, W)


# speedup vs baseline: 1.0987x; 1.0360x over previous
import functools

import jax
import jax.numpy as jnp
from jax import lax
from jax.experimental import pallas as pl
from jax.experimental.pallas import tpu as pltpu
from jax.experimental.pallas import tpu_sc as plsc

NC = 2
NS = 16
NW = NC * NS
CH = 16
ND = 4
ZW = 12


def _pad_rows(rows, n_pad_to):
    extra = n_pad_to - rows.shape[0]
    if extra == 0:
        return rows
    reps = -(-extra // rows.shape[0])
    return jnp.concatenate([rows, jnp.tile(rows, reps)[:extra]])


def _sc_body(nch_c, nch_z, row, xf, csrc, cdst, zdst, out,
             csrc_v, cdst_v, zdst_v, buf, zbuf, sem_z, sems_g, sems_s):
    wid = lax.axis_index("s") * NC + lax.axis_index("c")

    pltpu.sync_copy(csrc.at[wid], csrc_v)
    pltpu.sync_copy(cdst.at[wid], cdst_v)
    pltpu.sync_copy(zdst.at[wid], zdst_v)

    zeros16 = jnp.zeros((16,), jnp.float32)

    def zfill(t, carry):
        i = t // (row // 16)
        j = t % (row // 16)
        zbuf[i, pl.ds(j * 16, 16)] = zeros16
        return carry

    lax.fori_loop(0, CH * (row // 16), zfill, 0)

    def zero_fire(c):
        pltpu.async_copy(zbuf, out.at[zdst_v.at[c]], sem_z)

    def zero_drain(c):
        pltpu.make_async_copy(zbuf, out.at[zdst_v.at[c]], sem_z).wait()

    def gather_fire(c, j):
        pltpu.async_copy(xf.at[csrc_v.at[c]], buf.at[j], sems_g[j])

    def gather_wait(c, j):
        pltpu.make_async_copy(xf.at[csrc_v.at[c]], buf.at[j], sems_g[j]).wait()

    def scatter_fire(c, j):
        pltpu.async_copy(buf.at[j], out.at[cdst_v.at[c]], sems_s[j])

    def scatter_wait(c, j):
        pltpu.make_async_copy(buf.at[j], out.at[cdst_v.at[c]], sems_s[j]).wait()

    def copy_blocks(fused):
        nblk = nch_c // ND
        rem = nch_c % ND

        def blk_body(blk, carry):
            c0 = blk * ND
            for j in range(ND):
                @pl.when(blk > 0)
                def _(j=j):
                    scatter_wait(c0 - ND + j, j)
                gather_fire(c0 + j, j)
            if fused:
                for j in range(ND):
                    zero_fire(c0 + j)

                @pl.when(blk > 0)
                def _():
                    for j in range(ND):
                        zero_drain(c0 - ND + j)
            for j in range(ND):
                gather_wait(c0 + j, j)
                scatter_fire(c0 + j, j)
            return carry

        if nblk > 0:
            lax.fori_loop(0, nblk, blk_body, 0)

        for j in range(rem):
            c = nblk * ND + j
            if nblk > 0:
                scatter_wait(c - ND, j)
            gather_fire(c, j)
            if fused:
                zero_fire(c)
            gather_wait(c, j)
            scatter_fire(c, j)

        for j in range(ND if nblk > 0 else rem):
            cc = nblk * ND + j if j < rem else (nblk - 1) * ND + j
            scatter_wait(cc, j)

        if fused:
            start = (nblk - 1) * ND if nblk > 0 else 0

            def zdrain_loop(c, carry):
                zero_drain(c)
                return carry

            lax.fori_loop(start, nch_c, zdrain_loop, 0)

    if nch_c == nch_z:
        copy_blocks(fused=True)
    else:
        def zloop(c, carry):
            zero_fire(c)

            @pl.when(c >= ZW)
            def _():
                zero_drain(c - ZW)
            return carry

        lax.fori_loop(0, nch_z, zloop, 0)
        copy_blocks(fused=False)

        def zdrain_loop(c, carry):
            zero_drain(c)
            return carry

        lax.fori_loop(max(nch_z - ZW, 0), nch_z, zdrain_loop, 0)


def kernel(x, in_channels, out_channels):
    B, C_in, H, W = x.shape
    C_out = out_channels.shape[0]
    P = H * W
    row = 1024 if P % 1024 == 0 else P
    G = P // row

    def nonzero_positions(mask, count):
        mask_i = mask.astype(jnp.int32)
        rank = jnp.cumsum(mask_i) - 1
        sel = (rank[None, :] == jnp.arange(count, dtype=jnp.int32)[:, None])
        sel = sel & (mask_i[None, :] == 1)
        idx = jnp.arange(mask.shape[0], dtype=jnp.int32)
        return (sel.astype(jnp.int32) * idx[None, :]).sum(axis=1)

    nzo = out_channels != 0
    in_pos = nonzero_positions(in_channels != 0, C_in)
    out_pos = nonzero_positions(nzo, C_in)
    inv_pos = nonzero_positions(~nzo, C_out - C_in)

    b = jnp.arange(B, dtype=jnp.int32)
    g = jnp.arange(G, dtype=jnp.int32)

    def plane_rows(chan_pos, stride):
        planes = b[:, None] * stride + chan_pos[None, :].astype(jnp.int32)
        return (planes[:, :, None] * G + g[None, None, :]).reshape(-1)

    src_rows = plane_rows(in_pos, C_in)
    dst_rows = plane_rows(out_pos, C_out)
    zdst_rows = plane_rows(inv_pos, C_out)

    nch_c = -(-src_rows.shape[0] // (NW * CH))
    nch_z = -(-zdst_rows.shape[0] // (NW * CH))
    csrc = _pad_rows(src_rows, NW * nch_c * CH).reshape(NW, nch_c, CH)
    cdst = _pad_rows(dst_rows, NW * nch_c * CH).reshape(NW, nch_c, CH)
    zdst = _pad_rows(zdst_rows, NW * nch_z * CH).reshape(NW, nch_z, CH)

    xf = x.reshape(B * C_in * G, row)

    mesh = plsc.VectorSubcoreMesh(
        core_axis_name="c", subcore_axis_name="s",
        num_cores=NC, num_subcores=NS)
    body = functools.partial(_sc_body, nch_c, nch_z, row)
    out = pl.kernel(
        body,
        out_type=jax.ShapeDtypeStruct((B * C_out * G, row), jnp.float32),
        mesh=mesh,
        scratch_types=[
            pltpu.VMEM((nch_c, CH), jnp.int32),
            pltpu.VMEM((nch_c, CH), jnp.int32),
            pltpu.VMEM((nch_z, CH), jnp.int32),
            pltpu.VMEM((ND, CH, row), jnp.float32),
            pltpu.VMEM((CH, row), jnp.float32),
            pltpu.SemaphoreType.DMA,
            [pltpu.SemaphoreType.DMA] * ND,
            [pltpu.SemaphoreType.DMA] * ND,
        ],
    )(xf, csrc, cdst, zdst)
    return out.reshape(B, C_out, H, W)


# speedup vs baseline: 3.5812x; 3.2594x over previous
import functools

import jax
import jax.numpy as jnp
from jax import lax
from jax.experimental import pallas as pl
from jax.experimental.pallas import tpu as pltpu
from jax.experimental.pallas import tpu_sc as plsc

NC = 2
NS = 16
NW = NC * NS


def _pad_rows(rows, n_pad_to):
    extra = n_pad_to - rows.shape[0]
    if extra == 0:
        return rows
    reps = -(-extra // rows.shape[0])
    return jnp.concatenate([rows, jnp.tile(rows, reps)[:extra]])


def _sc_body(kc, kz, H, W, xf, csrc, cdst, zdst, out,
             csrc_v, cdst_v, zdst_v, b0, b1,
             sg0, sg1, ss0, ss1, semz):
    wid = lax.axis_index("s") * NC + lax.axis_index("c")

    pltpu.sync_copy(csrc.at[wid], csrc_v)
    pltpu.sync_copy(cdst.at[wid], cdst_v)
    pltpu.sync_copy(zdst.at[wid], zdst_v)
    csv = [csrc_v[pl.ds(16 * c, 16)] for c in range(-(-kc // 16))]
    cdv = [cdst_v[pl.ds(16 * c, 16)] for c in range(-(-kc // 16))]
    zdv = [zdst_v[pl.ds(16 * c, 16)] for c in range(-(-kz // 16))]

    def _idx(vs, k):
        return vs[k // 16][k % 16]

    zeros16 = jnp.zeros((16,), jnp.float32)
    nw16 = W // 16

    def zfill(t, carry):
        b0[0, t // nw16, pl.ds((t % nw16) * 16, 16)] = zeros16
        return carry

    lax.fori_loop(0, H * nw16, zfill, 0)

    def zero_fire(k):
        d = _idx(zdv, k)
        pltpu.async_copy(b0, out.at[pl.ds(d, 1)], semz)

    def zero_drain(k):
        d = _idx(zdv, k)
        pltpu.make_async_copy(b0, out.at[pl.ds(d, 1)], semz).wait()

    bufs = (b0, b1)
    gsems = (sg0, sg1)
    ssems = (ss0, ss1)

    def gather_fire(k):
        s = _idx(csv, k)
        pltpu.async_copy(xf.at[pl.ds(s, 1)], bufs[k % 2], gsems[k % 2])

    def gather_wait(k):
        s = _idx(csv, k)
        pltpu.make_async_copy(
            xf.at[pl.ds(s, 1)], bufs[k % 2], gsems[k % 2]).wait()

    def scatter_fire(k):
        d = _idx(cdv, k)
        pltpu.async_copy(bufs[k % 2], out.at[pl.ds(d, 1)], ssems[k % 2])

    def scatter_wait(k):
        d = _idx(cdv, k)
        pltpu.make_async_copy(
            bufs[k % 2], out.at[pl.ds(d, 1)], ssems[k % 2]).wait()

    for k in range(kz):
        zero_fire(k)
        if k >= 4:
            zero_drain(k - 4)
    for k in range(max(kz - 4, 0), kz):
        zero_drain(k)

    if kc > 0:
        gather_fire(0)
    for k in range(kc):
        if k >= 1:
            scatter_wait(k - 1)
        if k + 1 < kc:
            gather_fire(k + 1)
        gather_wait(k)
        scatter_fire(k)
    if kc > 0:
        scatter_wait(kc - 1)


def kernel(x, in_channels, out_channels):
    B, C_in, H, W = x.shape
    C_out = out_channels.shape[0]

    def nonzero_positions(mask, count):
        mask_i = mask.astype(jnp.int32)
        rank = jnp.cumsum(mask_i) - 1
        sel = (rank[None, :] == jnp.arange(count, dtype=jnp.int32)[:, None])
        sel = sel & (mask_i[None, :] == 1)
        idx = jnp.arange(mask.shape[0], dtype=jnp.int32)
        return (sel.astype(jnp.int32) * idx[None, :]).sum(axis=1)

    nzo = out_channels != 0
    in_pos = nonzero_positions(in_channels != 0, C_in)
    out_pos = nonzero_positions(nzo, C_in)
    inv_pos = nonzero_positions(~nzo, C_out - C_in)

    b = jnp.arange(B, dtype=jnp.int32)
    src_planes = (b[:, None] * C_in + in_pos[None, :]).reshape(-1)
    dst_planes = (b[:, None] * C_out + out_pos[None, :]).reshape(-1)
    zdst_planes = (b[:, None] * C_out + inv_pos[None, :]).reshape(-1)

    kc = -(-src_planes.shape[0] // NW)
    kz = -(-zdst_planes.shape[0] // NW)

    def worker_rows(rows, k):
        kp = -(-k // 128) * 128
        t = _pad_rows(rows, NW * k).reshape(NW, k)
        return jnp.pad(t, ((0, 0), (0, kp - k)))

    csrc = worker_rows(src_planes, kc)
    cdst = worker_rows(dst_planes, kc)
    zdst = worker_rows(zdst_planes, kz)

    xf = x.reshape(B * C_in, H, W)

    mesh = plsc.VectorSubcoreMesh(
        core_axis_name="c", subcore_axis_name="s",
        num_cores=NC, num_subcores=NS)
    body = functools.partial(_sc_body, kc, kz, H, W)
    out = pl.kernel(
        body,
        out_type=jax.ShapeDtypeStruct((B * C_out, H, W), jnp.float32),
        mesh=mesh,
        compiler_params=pltpu.CompilerParams(use_tc_tiling_on_sc=True),
        scratch_types=[
            pltpu.VMEM((-(-kc // 128) * 128,), jnp.int32),
            pltpu.VMEM((-(-kc // 128) * 128,), jnp.int32),
            pltpu.VMEM((-(-kz // 128) * 128,), jnp.int32),
            pltpu.VMEM((1, H, W), jnp.float32),
            pltpu.VMEM((1, H, W), jnp.float32),
            pltpu.SemaphoreType.DMA,
            pltpu.SemaphoreType.DMA,
            pltpu.SemaphoreType.DMA,
            pltpu.SemaphoreType.DMA,
            pltpu.SemaphoreType.DMA,
        ],
    )(xf, csrc, cdst, zdst)
    return out.reshape(B, C_out, H, W)


# speedup vs baseline: 3.7862x; 1.0572x over previous
import functools

import jax
import jax.numpy as jnp
from jax import lax
from jax.experimental import pallas as pl
from jax.experimental.pallas import tpu as pltpu
from jax.experimental.pallas import tpu_sc as plsc

NC = 2
NS = 16
NW = NC * NS


def _pad_rows(rows, n_pad_to):
    extra = n_pad_to - rows.shape[0]
    if extra == 0:
        return rows
    reps = -(-extra // rows.shape[0])
    return jnp.concatenate([rows, jnp.tile(rows, reps)[:extra]])


def _sc_body(kc, kz, H, W, xf, csrc, cdst, zdst, out,
             csrc_v, cdst_v, zdst_v, b0, b1, zb,
             sg0, sg1, ss0, ss1, semz):
    wid = lax.axis_index("s") * NC + lax.axis_index("c")

    pltpu.sync_copy(csrc.at[wid], csrc_v)
    pltpu.sync_copy(cdst.at[wid], cdst_v)
    pltpu.sync_copy(zdst.at[wid], zdst_v)
    csv = [csrc_v[pl.ds(16 * c, 16)] for c in range(-(-kc // 16))]
    cdv = [cdst_v[pl.ds(16 * c, 16)] for c in range(-(-kc // 16))]
    zdv = [zdst_v[pl.ds(16 * c, 16)] for c in range(-(-kz // 16))]

    def _idx(vs, k):
        return vs[k // 16][k % 16]

    zeros16 = jnp.zeros((16,), jnp.float32)
    nw16 = W // 16
    HQ = H // 4

    def zfill(t, carry):
        zb[0, t // nw16, pl.ds((t % nw16) * 16, 16)] = zeros16
        return carry

    lax.fori_loop(0, HQ * nw16, zfill, 0)

    def zero_fire(k, q):
        d = _idx(zdv, k)
        pltpu.async_copy(zb, out.at[pl.ds(d, 1), pl.ds(q * HQ, HQ)], semz)

    def zero_drain(k, q):
        d = _idx(zdv, k)
        pltpu.make_async_copy(
            zb, out.at[pl.ds(d, 1), pl.ds(q * HQ, HQ)], semz).wait()

    bufs = (b0, b1)
    gsems = (sg0, sg1)
    ssems = (ss0, ss1)

    def gather_fire(k):
        s = _idx(csv, k)
        pltpu.async_copy(xf.at[pl.ds(s, 1)], bufs[k % 2], gsems[k % 2])

    def gather_wait(k):
        s = _idx(csv, k)
        pltpu.make_async_copy(
            xf.at[pl.ds(s, 1)], bufs[k % 2], gsems[k % 2]).wait()

    def scatter_fire(k):
        d = _idx(cdv, k)
        pltpu.async_copy(bufs[k % 2], out.at[pl.ds(d, 1)], ssems[k % 2])

    def scatter_wait(k):
        d = _idx(cdv, k)
        pltpu.make_async_copy(
            bufs[k % 2], out.at[pl.ds(d, 1)], ssems[k % 2]).wait()

    if kc > 0:
        gather_fire(0)
    for k in range(kc):
        if k >= 1:
            scatter_wait(k - 1)
        if k + 1 < kc:
            gather_fire(k + 1)
        if k < kz:
            for q in range(4):
                zero_fire(k, q)
        if 2 <= k and k - 2 < kz:
            for q in range(4):
                zero_drain(k - 2, q)
        gather_wait(k)
        scatter_fire(k)
    if kc > 0:
        scatter_wait(kc - 1)
    for k in range(kc, kz):
        for q in range(4):
            zero_fire(k, q)
    for k in range(max(min(kc - 2, kz), 0), kz):
        for q in range(4):
            zero_drain(k, q)


def kernel(x, in_channels, out_channels):
    B, C_in, H, W = x.shape
    C_out = out_channels.shape[0]

    def nonzero_positions(mask, count):
        mask_i = mask.astype(jnp.int32)
        rank = jnp.cumsum(mask_i) - 1
        sel = (rank[None, :] == jnp.arange(count, dtype=jnp.int32)[:, None])
        sel = sel & (mask_i[None, :] == 1)
        idx = jnp.arange(mask.shape[0], dtype=jnp.int32)
        return (sel.astype(jnp.int32) * idx[None, :]).sum(axis=1)

    nzo = out_channels != 0
    in_pos = nonzero_positions(in_channels != 0, C_in)
    out_pos = nonzero_positions(nzo, C_in)
    inv_pos = nonzero_positions(~nzo, C_out - C_in)

    b = jnp.arange(B, dtype=jnp.int32)
    src_planes = (b[:, None] * C_in + in_pos[None, :]).reshape(-1)
    dst_planes = (b[:, None] * C_out + out_pos[None, :]).reshape(-1)
    zdst_planes = (b[:, None] * C_out + inv_pos[None, :]).reshape(-1)

    kc = -(-src_planes.shape[0] // NW)
    kz = -(-zdst_planes.shape[0] // NW)

    def worker_rows(rows, k):
        kp = -(-k // 128) * 128
        t = _pad_rows(rows, NW * k).reshape(NW, k)
        return jnp.pad(t, ((0, 0), (0, kp - k)))

    csrc = worker_rows(src_planes, kc)
    cdst = worker_rows(dst_planes, kc)
    zdst = worker_rows(zdst_planes, kz)

    xf = x.reshape(B * C_in, H, W)

    mesh = plsc.VectorSubcoreMesh(
        core_axis_name="c", subcore_axis_name="s",
        num_cores=NC, num_subcores=NS)
    body = functools.partial(_sc_body, kc, kz, H, W)
    out = pl.kernel(
        body,
        out_type=jax.ShapeDtypeStruct((B * C_out, H, W), jnp.float32),
        mesh=mesh,
        compiler_params=pltpu.CompilerParams(use_tc_tiling_on_sc=True),
        scratch_types=[
            pltpu.VMEM((-(-kc // 128) * 128,), jnp.int32),
            pltpu.VMEM((-(-kc // 128) * 128,), jnp.int32),
            pltpu.VMEM((-(-kz // 128) * 128,), jnp.int32),
            pltpu.VMEM((1, H, W), jnp.float32),
            pltpu.VMEM((1, H, W), jnp.float32),
            pltpu.VMEM((1, H // 4, W), jnp.float32),
            pltpu.SemaphoreType.DMA,
            pltpu.SemaphoreType.DMA,
            pltpu.SemaphoreType.DMA,
            pltpu.SemaphoreType.DMA,
            pltpu.SemaphoreType.DMA,
        ],
    )(xf, csrc, cdst, zdst)
    return out.reshape(B, C_out, H, W)


# speedup vs baseline: 3.8935x; 1.0283x over previous
import functools

import jax
import jax.numpy as jnp
from jax import lax
from jax.experimental import pallas as pl
from jax.experimental.pallas import tpu as pltpu
from jax.experimental.pallas import tpu_sc as plsc

NC = 2
NS = 16
NW = NC * NS


def _pad_rows(rows, n_pad_to):
    extra = n_pad_to - rows.shape[0]
    if extra == 0:
        return rows
    reps = -(-extra // rows.shape[0])
    return jnp.concatenate([rows, jnp.tile(rows, reps)[:extra]])


def _sc_body(kc, kz, H, W, xf, tab, out,
             csrc_v, cdst_v, zdst_v, b0, b1, zb,
             sg0, sg1, ss0, ss1, semz):
    wid = lax.axis_index("s") * NC + lax.axis_index("c")

    pltpu.sync_copy(tab.at[0, wid], csrc_v)
    pltpu.sync_copy(tab.at[1, wid], cdst_v)
    pltpu.sync_copy(tab.at[2, wid], zdst_v)
    csv = [csrc_v[pl.ds(16 * c, 16)] for c in range(-(-kc // 16))]
    cdv = [cdst_v[pl.ds(16 * c, 16)] for c in range(-(-kc // 16))]
    zdv = [zdst_v[pl.ds(16 * c, 16)] for c in range(-(-kz // 16))]

    def _idx(vs, k):
        return vs[k // 16][k % 16]

    zeros16 = jnp.zeros((16,), jnp.float32)
    nw16 = W // 16
    HQ = H // 4

    def zero_fire(k, q):
        d = _idx(zdv, k)
        pltpu.async_copy(zb, out.at[pl.ds(d, 1), pl.ds(q * HQ, HQ)], semz)

    def zero_drain(k, q):
        d = _idx(zdv, k)
        pltpu.make_async_copy(
            zb, out.at[pl.ds(d, 1), pl.ds(q * HQ, HQ)], semz).wait()

    bufs = (b0, b1)
    gsems = (sg0, sg1)
    ssems = (ss0, ss1)

    def gather_fire(k):
        s = _idx(csv, k)
        pltpu.async_copy(xf.at[pl.ds(s, 1)], bufs[k % 2], gsems[k % 2])

    def gather_wait(k):
        s = _idx(csv, k)
        pltpu.make_async_copy(
            xf.at[pl.ds(s, 1)], bufs[k % 2], gsems[k % 2]).wait()

    def scatter_fire(k):
        d = _idx(cdv, k)
        pltpu.async_copy(bufs[k % 2], out.at[pl.ds(d, 1)], ssems[k % 2])

    def scatter_wait(k):
        d = _idx(cdv, k)
        pltpu.make_async_copy(
            bufs[k % 2], out.at[pl.ds(d, 1)], ssems[k % 2]).wait()

    if kc > 0:
        gather_fire(0)

    def zfill(t, carry):
        zb[0, t // nw16, pl.ds((t % nw16) * 16, 16)] = zeros16
        return carry

    lax.fori_loop(0, HQ * nw16, zfill, 0)

    for k in range(kc):
        if k >= 1:
            scatter_wait(k - 1)
        if k + 1 < kc:
            gather_fire(k + 1)
        if k < kz:
            for q in range(4):
                zero_fire(k, q)
        if 2 <= k and k - 2 < kz:
            for q in range(4):
                zero_drain(k - 2, q)
        gather_wait(k)
        scatter_fire(k)
    if kc > 0:
        scatter_wait(kc - 1)
    for k in range(kc, kz):
        for q in range(4):
            zero_fire(k, q)
    for k in range(max(min(kc - 2, kz), 0), kz):
        for q in range(4):
            zero_drain(k, q)


def kernel(x, in_channels, out_channels):
    B, C_in, H, W = x.shape
    C_out = out_channels.shape[0]

    jmax = max(C_in, C_out - C_in)
    io = jnp.arange(C_out, dtype=jnp.int32)
    m_in = jnp.pad((in_channels != 0).astype(jnp.int32), (0, C_out - C_in))
    m_out = (out_channels != 0).astype(jnp.int32)
    cs_in = jnp.cumsum(m_in) - 1
    cs_out = jnp.cumsum(m_out) - 1
    ranks3 = jnp.stack([cs_in, cs_out, io - cs_out - 1])
    masks3 = jnp.stack([m_in, m_out, 1 - m_out])
    j = jnp.arange(jmax, dtype=jnp.int32)
    sel = (ranks3[:, None, :] == j[None, :, None]) & (masks3[:, None, :] == 1)
    pos3 = (sel.astype(jnp.int32) * io[None, None, :]).sum(-1)

    b = jnp.arange(B, dtype=jnp.int32)
    strides3 = jnp.array([C_in, C_out, C_out], dtype=jnp.int32)
    planes3 = (b[None, :, None] * strides3[:, None, None]
               + pos3[:, None, :])

    kc = -(-(B * C_in) // NW)
    kz = -(-(B * (C_out - C_in)) // NW)
    kp = -(-max(kc, kz) // 128) * 128

    def worker_rows(rows, k):
        t = _pad_rows(rows, NW * k).reshape(NW, k)
        return jnp.pad(t, ((0, 0), (0, kp - k)))

    tab = jnp.stack([
        worker_rows(planes3[0, :, :C_in].reshape(-1), kc),
        worker_rows(planes3[1, :, :C_in].reshape(-1), kc),
        worker_rows(planes3[2, :, :C_out - C_in].reshape(-1), kz),
    ])

    xf = x.reshape(B * C_in, H, W)

    mesh = plsc.VectorSubcoreMesh(
        core_axis_name="c", subcore_axis_name="s",
        num_cores=NC, num_subcores=NS)
    body = functools.partial(_sc_body, kc, kz, H, W)
    out = pl.kernel(
        body,
        out_type=jax.ShapeDtypeStruct((B * C_out, H, W), jnp.float32),
        mesh=mesh,
        compiler_params=pltpu.CompilerParams(use_tc_tiling_on_sc=True),
        scratch_types=[
            pltpu.VMEM((kp,), jnp.int32),
            pltpu.VMEM((kp,), jnp.int32),
            pltpu.VMEM((kp,), jnp.int32),
            pltpu.VMEM((1, H, W), jnp.float32),
            pltpu.VMEM((1, H, W), jnp.float32),
            pltpu.VMEM((1, H // 4, W), jnp.float32),
            pltpu.SemaphoreType.DMA,
            pltpu.SemaphoreType.DMA,
            pltpu.SemaphoreType.DMA,
            pltpu.SemaphoreType.DMA,
            pltpu.SemaphoreType.DMA,
        ],
    )(xf, tab)
    return out.reshape(B, C_out, H, W)
